# baseline (speedup 1.0000x reference)
import functools
import jax
import jax.numpy as jnp
from jax import lax
from jax.experimental import pallas as pl
from jax.experimental.pallas import tpu as pltpu
from jax.experimental.pallas import tpu_sc as plsc

B = 1024
N = 100000
D_IN = 64
D_MAIN = 128
D_BLOCK = 256
C = 96
EPS = 1e-5

N_PAD = 100352
ENC_BLK = 2048
SCORE_QB = 128
ATT_QB = 64
NEG = -1e30


def _encode_cand_kernel(x_ref, wl_ref, bl_ref, wk_ref, bk_ref, ck_ref):
    h = jnp.dot(x_ref[...], wl_ref[...], preferred_element_type=jnp.float32)
    h = h + bl_ref[...]
    ck = jnp.dot(h, wk_ref[...], preferred_element_type=jnp.float32)
    ck_ref[...] = ck + bk_ref[...]


def _encode_query_kernel(x_ref, wl_ref, bl_ref, wk_ref, bk_ref, xq_ref, kq_ref):
    h = jnp.dot(x_ref[...], wl_ref[...], preferred_element_type=jnp.float32)
    h = h + bl_ref[...]
    xq_ref[...] = h
    kq_ref[...] = jnp.dot(h, wk_ref[...], preferred_element_type=jnp.float32) + bk_ref[...]


def _scores_kernel(kq_ref, ck_ref, s_ref):
    j = pl.program_id(1)
    ck = ck_ref[...]
    nsq = jnp.sum(ck * ck, axis=1)[None, :]
    s = 2.0 * jax.lax.dot_general(
        kq_ref[...], ck, (((1,), (1,)), ((), ())),
        preferred_element_type=jnp.float32) - nsq
    col = j * ENC_BLK + lax.broadcasted_iota(jnp.int32, s.shape, 1)
    s_ref[...] = jnp.where(col < N, s, NEG)


def _attn_kernel(xq_ref, kq_ref, ctxk_ref, ctxy_ref,
                 wlab_ref, blab_ref, wt1_ref, bt1_ref, wt2_ref,
                 ln1g_ref, ln1b_ref, wb1_ref, bb1_ref, wb2_ref, bb2_ref,
                 lnhg_ref, lnhb_ref, whead_ref, bhead_ref, out_ref):
    nq = xq_ref.shape[0]
    kq = kq_ref[...]
    ctxk = ctxk_ref[...]
    kq_rows = jnp.repeat(kq, C, axis=0)
    diff = kq_rows - ctxk

    sim = -jnp.sum(diff * diff, axis=1, keepdims=True)
    sim3 = sim.reshape(nq, C, 1)
    m = jnp.max(sim3, axis=1, keepdims=True)
    e = jnp.exp(sim3 - m)
    denom = jnp.sum(e, axis=1, keepdims=True)
    probs = e / denom

    y_emb = ctxy_ref[...] * wlab_ref[...] + blab_ref[...]

    t = jnp.dot(diff, wt1_ref[...], preferred_element_type=jnp.float32) + bt1_ref[...]
    t = jnp.maximum(t, 0.0)
    t = jnp.dot(t, wt2_ref[...], preferred_element_type=jnp.float32)

    values = (y_emb + t).reshape(nq, C, D_MAIN)
    ctx_x = jnp.sum(values * probs, axis=1)

    x = xq_ref[...] + ctx_x

    mu = jnp.mean(x, axis=1, keepdims=True)
    var = jnp.mean((x - mu) ** 2, axis=1, keepdims=True)
    h = (x - mu) * lax.rsqrt(var + EPS) * ln1g_ref[...] + ln1b_ref[...]
    h = jnp.dot(h, wb1_ref[...], preferred_element_type=jnp.float32) + bb1_ref[...]
    h = jnp.maximum(h, 0.0)
    x = x + jnp.dot(h, wb2_ref[...], preferred_element_type=jnp.float32) + bb2_ref[...]

    mu = jnp.mean(x, axis=1, keepdims=True)
    var = jnp.mean((x - mu) ** 2, axis=1, keepdims=True)
    h = (x - mu) * lax.rsqrt(var + EPS) * lnhg_ref[...] + lnhb_ref[...]
    h = jnp.maximum(h, 0.0)
    out_ref[...] = jnp.dot(h, whead_ref[...], preferred_element_type=jnp.float32) + bhead_ref[...]


def _full(shape):
    return pl.BlockSpec(shape, lambda *_: tuple(0 for _ in shape))


GATHER_CHUNK = 384


def _sc_gather(ck_hbm, y_hbm, idx_hbm):
    info = plsc.get_sparse_core_info()
    nw = info.num_cores * info.num_subcores
    b_per_w = (B * C) // nw
    nchunks = b_per_w // GATHER_CHUNK

    mesh = plsc.VectorSubcoreMesh(core_axis_name="c", subcore_axis_name="s")

    @functools.partial(
        pl.kernel,
        mesh=mesh,
        out_type=[jax.ShapeDtypeStruct((B * C, D_MAIN), jnp.float32),
                  jax.ShapeDtypeStruct((B * C,), jnp.float32)],
        scratch_types=[
            pltpu.VMEM((b_per_w,), jnp.int32),
            pltpu.VMEM((GATHER_CHUNK, D_MAIN), jnp.float32),
            pltpu.VMEM((GATHER_CHUNK,), jnp.float32),
            pltpu.SemaphoreType.DMA,
            pltpu.SemaphoreType.DMA,
        ],
    )
    def k(ck_ref, y_ref, idx_ref, outk_ref, outy_ref,
          idx_v, rows_v, yrows_v, sem1, sem2):
        wid = lax.axis_index("s") * info.num_cores + lax.axis_index("c")
        base = wid * b_per_w
        pltpu.sync_copy(idx_ref.at[pl.ds(base, b_per_w)], idx_v)

        def body(c, _):
            off = c * GATHER_CHUNK
            cidx = idx_v.at[pl.ds(off, GATHER_CHUNK)]
            pltpu.async_copy(ck_ref.at[cidx], rows_v, sem1).wait()
            pltpu.sync_copy(rows_v, outk_ref.at[pl.ds(base + off, GATHER_CHUNK)])
            pltpu.async_copy(y_ref.at[cidx], yrows_v, sem2).wait()
            pltpu.sync_copy(yrows_v, outy_ref.at[pl.ds(base + off, GATHER_CHUNK)])
            return ()

        lax.fori_loop(0, nchunks, body, ())

    return k(ck_hbm, y_hbm, idx_hbm)


def kernel(x_num, candidate_x_num, candidate_y, W_lin, b_lin, W_K, b_K,
           W_label, b_label, W_T1, b_T1, W_T2, ln1_g, ln1_b,
           W_b1, b_b1, W_b2, b_b2, lnh_g, lnh_b, W_head, b_head):
    f32 = jnp.float32

    cand_pad = jnp.pad(candidate_x_num, ((0, N_PAD - N), (0, 0)))
    ck = pl.pallas_call(
        _encode_cand_kernel,
        grid=(N_PAD // ENC_BLK,),
        in_specs=[
            pl.BlockSpec((ENC_BLK, D_IN), lambda i: (i, 0)),
            _full((D_IN, D_MAIN)),
            _full((D_MAIN,)),
            _full((D_MAIN, D_MAIN)),
            _full((D_MAIN,)),
        ],
        out_specs=pl.BlockSpec((ENC_BLK, D_MAIN), lambda i: (i, 0)),
        out_shape=jax.ShapeDtypeStruct((N_PAD, D_MAIN), f32),
    )(cand_pad, W_lin, b_lin, W_K, b_K)

    xq, kq = pl.pallas_call(
        _encode_query_kernel,
        grid=(1,),
        in_specs=[
            _full((B, D_IN)),
            _full((D_IN, D_MAIN)),
            _full((D_MAIN,)),
            _full((D_MAIN, D_MAIN)),
            _full((D_MAIN,)),
        ],
        out_specs=[_full((B, D_MAIN)), _full((B, D_MAIN))],
        out_shape=[jax.ShapeDtypeStruct((B, D_MAIN), f32),
                   jax.ShapeDtypeStruct((B, D_MAIN), f32)],
    )(x_num, W_lin, b_lin, W_K, b_K)

    scores = pl.pallas_call(
        _scores_kernel,
        grid=(B // SCORE_QB, N_PAD // ENC_BLK),
        in_specs=[
            pl.BlockSpec((SCORE_QB, D_MAIN), lambda i, j: (i, 0)),
            pl.BlockSpec((ENC_BLK, D_MAIN), lambda i, j: (j, 0)),
        ],
        out_specs=pl.BlockSpec((SCORE_QB, ENC_BLK), lambda i, j: (i, j)),
        out_shape=jax.ShapeDtypeStruct((B, N_PAD), f32),
    )(kq, ck)

    _, context_idx = lax.top_k(scores, C)

    y_pad = jnp.pad(candidate_y, (0, N_PAD - N))
    idx_flat = context_idx.reshape(-1)
    ctx_k, ctx_yflat = _sc_gather(ck, y_pad, idx_flat)
    ctx_y = ctx_yflat[:, None]

    W_head_pad = jnp.pad(W_head, ((0, 0), (0, D_MAIN - 2)))
    b_head_pad = jnp.pad(b_head, (0, D_MAIN - 2))
    out = pl.pallas_call(
        _attn_kernel,
        grid=(B // ATT_QB,),
        in_specs=[
            pl.BlockSpec((ATT_QB, D_MAIN), lambda i: (i, 0)),
            pl.BlockSpec((ATT_QB, D_MAIN), lambda i: (i, 0)),
            pl.BlockSpec((ATT_QB * C, D_MAIN), lambda i: (i, 0)),
            pl.BlockSpec((ATT_QB * C, 1), lambda i: (i, 0)),
            _full((1, D_MAIN)),
            _full((D_MAIN,)),
            _full((D_MAIN, D_BLOCK)),
            _full((D_BLOCK,)),
            _full((D_BLOCK, D_MAIN)),
            _full((D_MAIN,)),
            _full((D_MAIN,)),
            _full((D_MAIN, D_BLOCK)),
            _full((D_BLOCK,)),
            _full((D_BLOCK, D_MAIN)),
            _full((D_MAIN,)),
            _full((D_MAIN,)),
            _full((D_MAIN,)),
            _full((D_MAIN, D_MAIN)),
            _full((D_MAIN,)),
        ],
        out_specs=pl.BlockSpec((ATT_QB, D_MAIN), lambda i: (i, 0)),
        out_shape=jax.ShapeDtypeStruct((B, D_MAIN), f32),
    )(xq, kq, ctx_k, ctx_y, W_label, b_label, W_T1, b_T1, W_T2,
      ln1_g, ln1_b, W_b1, b_b1, W_b2, b_b2, lnh_g, lnh_b,
      W_head_pad, b_head_pad)

    return out[:, :2]


# speedup vs baseline: 5.4899x; 5.4899x over previous
import functools
import jax
import jax.numpy as jnp
from jax import lax
from jax.experimental import pallas as pl
from jax.experimental.pallas import tpu as pltpu
from jax.experimental.pallas import tpu_sc as plsc

B = 1024
N = 100000
D_IN = 64
D_MAIN = 128
D_BLOCK = 256
C = 96
EPS = 1e-5

N_PAD = 100352
ENC_BLK = 2048
SCORE_QB = 128
ATT_QB = 64
NEG = -1e30
GRP = 128
NGRP = N_PAD // GRP


def _encode_cand_kernel(x_ref, wl_ref, bl_ref, wk_ref, bk_ref, ck_ref):
    h = jnp.dot(x_ref[...], wl_ref[...], preferred_element_type=jnp.float32)
    h = h + bl_ref[...]
    ck = jnp.dot(h, wk_ref[...], preferred_element_type=jnp.float32)
    ck_ref[...] = ck + bk_ref[...]


def _encode_query_kernel(x_ref, wl_ref, bl_ref, wk_ref, bk_ref, xq_ref, kq_ref):
    h = jnp.dot(x_ref[...], wl_ref[...], preferred_element_type=jnp.float32)
    h = h + bl_ref[...]
    xq_ref[...] = h
    kq_ref[...] = jnp.dot(h, wk_ref[...], preferred_element_type=jnp.float32) + bk_ref[...]


def _scores_kernel(kq_ref, ck_ref, s_ref, l2_ref):
    j = pl.program_id(1)
    ck = ck_ref[...]
    nsq = jnp.sum(ck * ck, axis=1)[None, :]
    s = 2.0 * jax.lax.dot_general(
        kq_ref[...], ck, (((1,), (1,)), ((), ())),
        preferred_element_type=jnp.float32) - nsq
    col = j * ENC_BLK + lax.broadcasted_iota(jnp.int32, s.shape, 1)
    s = jnp.where(col < N, s, NEG)
    v = lax.bitcast_convert_type(s, jnp.int32)
    key = v ^ ((v >> 31) & jnp.int32(0x7FFFFFFF))
    s_ref[...] = key
    l2_ref[...] = jnp.max(key.reshape(SCORE_QB, ENC_BLK // GRP, GRP), axis=2)[None]


def _attn_kernel(xq_ref, kq_ref, ctxk_ref, ctxy_ref,
                 wlab_ref, blab_ref, wt1_ref, bt1_ref, wt2_ref,
                 ln1g_ref, ln1b_ref, wb1_ref, bb1_ref, wb2_ref, bb2_ref,
                 lnhg_ref, lnhb_ref, whead_ref, bhead_ref, out_ref):
    nq = xq_ref.shape[0]
    kq = kq_ref[...]
    ctxk = ctxk_ref[...]
    kq_rows = jnp.repeat(kq, C, axis=0)
    diff = kq_rows - ctxk

    sim = -jnp.sum(diff * diff, axis=1, keepdims=True)
    sim3 = sim.reshape(nq, C, 1)
    m = jnp.max(sim3, axis=1, keepdims=True)
    e = jnp.exp(sim3 - m)
    denom = jnp.sum(e, axis=1, keepdims=True)
    probs = e / denom

    y_emb = ctxy_ref[...] * wlab_ref[...] + blab_ref[...]

    t = jnp.dot(diff, wt1_ref[...], preferred_element_type=jnp.float32) + bt1_ref[...]
    t = jnp.maximum(t, 0.0)
    t = jnp.dot(t, wt2_ref[...], preferred_element_type=jnp.float32)

    values = (y_emb + t).reshape(nq, C, D_MAIN)
    ctx_x = jnp.sum(values * probs, axis=1)

    x = xq_ref[...] + ctx_x

    mu = jnp.mean(x, axis=1, keepdims=True)
    var = jnp.mean((x - mu) ** 2, axis=1, keepdims=True)
    h = (x - mu) * lax.rsqrt(var + EPS) * ln1g_ref[...] + ln1b_ref[...]
    h = jnp.dot(h, wb1_ref[...], preferred_element_type=jnp.float32) + bb1_ref[...]
    h = jnp.maximum(h, 0.0)
    x = x + jnp.dot(h, wb2_ref[...], preferred_element_type=jnp.float32) + bb2_ref[...]

    mu = jnp.mean(x, axis=1, keepdims=True)
    var = jnp.mean((x - mu) ** 2, axis=1, keepdims=True)
    h = (x - mu) * lax.rsqrt(var + EPS) * lnhg_ref[...] + lnhb_ref[...]
    h = jnp.maximum(h, 0.0)
    out_ref[...] = jnp.dot(h, whead_ref[...], preferred_element_type=jnp.float32) + bhead_ref[...]


def _full(shape):
    return pl.BlockSpec(shape, lambda *_: tuple(0 for _ in shape))


def _extract96_kernel(keys_ref, lanes_ref, *, w, pre_reduce):
    v = keys_ref[...]
    nq = v.shape[0]
    if pre_reduce > 1:
        v = jnp.max(v.reshape(nq, w, pre_reduce), axis=2)
    cols = lax.broadcasted_iota(jnp.int32, (nq, w), 1)
    lanes0 = lax.broadcasted_iota(jnp.int32, (nq, 128), 1)
    big = jnp.int32(2147483647)
    neg = jnp.int32(-2147483648)

    def it(t, carry):
        vv, acc = carry
        m = jnp.max(vv, axis=1, keepdims=True)
        cand = jnp.where(vv == m, cols, big)
        j = jnp.min(cand, axis=1, keepdims=True)
        acc = jnp.where(lanes0 == t, j, acc)
        vv = jnp.where(cols == j, neg, vv)
        return (vv, acc)

    _, acc = lax.fori_loop(0, C, it, (v, jnp.zeros((nq, 128), jnp.int32)))
    lanes_ref[...] = acc


def _extract96(keys, w, pre_reduce=1, qb=128):
    lanes = pl.pallas_call(
        functools.partial(_extract96_kernel, w=w, pre_reduce=pre_reduce),
        grid=(B // qb,),
        in_specs=[pl.BlockSpec((qb, w * pre_reduce), lambda i: (i, 0))],
        out_specs=pl.BlockSpec((qb, 128), lambda i: (i, 0)),
        out_shape=jax.ShapeDtypeStruct((B, 128), jnp.int32),
    )(keys)
    return lanes[:, :C]


def _sc_gather_rows(table, idx, d, chunk):
    m = idx.shape[0]
    info = plsc.get_sparse_core_info()
    nw = info.num_cores * info.num_subcores
    b_per_w = m // nw
    nchunks = b_per_w // chunk
    mesh = plsc.VectorSubcoreMesh(core_axis_name="c", subcore_axis_name="s")
    out_sh = (m, d) if d else (m,)
    row_sh = (chunk, d) if d else (chunk,)

    @functools.partial(
        pl.kernel,
        mesh=mesh,
        out_type=jax.ShapeDtypeStruct(out_sh, table.dtype),
        scratch_types=[
            pltpu.VMEM((b_per_w,), jnp.int32),
            pltpu.VMEM(row_sh, table.dtype),
            pltpu.SemaphoreType.DMA,
        ],
    )
    def k(t_ref, idx_ref, out_ref, idx_v, rows_v, sem):
        wid = lax.axis_index("s") * info.num_cores + lax.axis_index("c")
        base = wid * b_per_w
        pltpu.sync_copy(idx_ref.at[pl.ds(base, b_per_w)], idx_v)

        def body(c, _):
            off = c * chunk
            cidx = idx_v.at[pl.ds(off, chunk)]
            pltpu.async_copy(t_ref.at[cidx], rows_v, sem).wait()
            pltpu.sync_copy(rows_v, out_ref.at[pl.ds(base + off, chunk)])
            return ()

        lax.fori_loop(0, nchunks, body, ())

    return k(table, idx)


GATHER_CHUNK = 384


def _sc_gather(ck_hbm, y_hbm, idx_hbm):
    info = plsc.get_sparse_core_info()
    nw = info.num_cores * info.num_subcores
    b_per_w = (B * C) // nw
    nchunks = b_per_w // GATHER_CHUNK

    mesh = plsc.VectorSubcoreMesh(core_axis_name="c", subcore_axis_name="s")

    @functools.partial(
        pl.kernel,
        mesh=mesh,
        out_type=[jax.ShapeDtypeStruct((B * C, D_MAIN), jnp.float32),
                  jax.ShapeDtypeStruct((B * C,), jnp.float32)],
        scratch_types=[
            pltpu.VMEM((b_per_w,), jnp.int32),
            pltpu.VMEM((GATHER_CHUNK, D_MAIN), jnp.float32),
            pltpu.VMEM((GATHER_CHUNK,), jnp.float32),
            pltpu.SemaphoreType.DMA,
            pltpu.SemaphoreType.DMA,
        ],
    )
    def k(ck_ref, y_ref, idx_ref, outk_ref, outy_ref,
          idx_v, rows_v, yrows_v, sem1, sem2):
        wid = lax.axis_index("s") * info.num_cores + lax.axis_index("c")
        base = wid * b_per_w
        pltpu.sync_copy(idx_ref.at[pl.ds(base, b_per_w)], idx_v)

        def body(c, _):
            off = c * GATHER_CHUNK
            cidx = idx_v.at[pl.ds(off, GATHER_CHUNK)]
            pltpu.async_copy(ck_ref.at[cidx], rows_v, sem1).wait()
            pltpu.sync_copy(rows_v, outk_ref.at[pl.ds(base + off, GATHER_CHUNK)])
            pltpu.async_copy(y_ref.at[cidx], yrows_v, sem2).wait()
            pltpu.sync_copy(yrows_v, outy_ref.at[pl.ds(base + off, GATHER_CHUNK)])
            return ()

        lax.fori_loop(0, nchunks, body, ())

    return k(ck_hbm, y_hbm, idx_hbm)


def kernel(x_num, candidate_x_num, candidate_y, W_lin, b_lin, W_K, b_K,
           W_label, b_label, W_T1, b_T1, W_T2, ln1_g, ln1_b,
           W_b1, b_b1, W_b2, b_b2, lnh_g, lnh_b, W_head, b_head):
    f32 = jnp.float32

    cand_pad = jnp.pad(candidate_x_num, ((0, N_PAD - N), (0, 0)))
    ck = pl.pallas_call(
        _encode_cand_kernel,
        grid=(N_PAD // ENC_BLK,),
        in_specs=[
            pl.BlockSpec((ENC_BLK, D_IN), lambda i: (i, 0)),
            _full((D_IN, D_MAIN)),
            _full((D_MAIN,)),
            _full((D_MAIN, D_MAIN)),
            _full((D_MAIN,)),
        ],
        out_specs=pl.BlockSpec((ENC_BLK, D_MAIN), lambda i: (i, 0)),
        out_shape=jax.ShapeDtypeStruct((N_PAD, D_MAIN), f32),
    )(cand_pad, W_lin, b_lin, W_K, b_K)

    xq, kq = pl.pallas_call(
        _encode_query_kernel,
        grid=(1,),
        in_specs=[
            _full((B, D_IN)),
            _full((D_IN, D_MAIN)),
            _full((D_MAIN,)),
            _full((D_MAIN, D_MAIN)),
            _full((D_MAIN,)),
        ],
        out_specs=[_full((B, D_MAIN)), _full((B, D_MAIN))],
        out_shape=[jax.ShapeDtypeStruct((B, D_MAIN), f32),
                   jax.ShapeDtypeStruct((B, D_MAIN), f32)],
    )(x_num, W_lin, b_lin, W_K, b_K)

    scores, l2max = pl.pallas_call(
        _scores_kernel,
        grid=(B // SCORE_QB, N_PAD // ENC_BLK),
        in_specs=[
            pl.BlockSpec((SCORE_QB, D_MAIN), lambda i, j: (i, 0)),
            pl.BlockSpec((ENC_BLK, D_MAIN), lambda i, j: (j, 0)),
        ],
        out_specs=[
            pl.BlockSpec((SCORE_QB, ENC_BLK), lambda i, j: (i, j)),
            pl.BlockSpec((1, SCORE_QB, ENC_BLK // GRP), lambda i, j: (j, i, 0)),
        ],
        out_shape=[jax.ShapeDtypeStruct((B, N_PAD), jnp.int32),
                   jax.ShapeDtypeStruct((N_PAD // ENC_BLK, B, ENC_BLK // GRP), jnp.int32)],
    )(kq, ck)
    l2max = jnp.transpose(l2max, (1, 0, 2)).reshape(B, NGRP)

    ar8 = jnp.arange(8, dtype=jnp.int32)
    qrow = jnp.arange(B, dtype=jnp.int32)[:, None]
    gid2 = _extract96(l2max, NGRP)
    sblk = _sc_gather_rows(scores.reshape(B * NGRP, GRP),
                           (gid2 + qrow * NGRP).reshape(-1), GRP, 256)
    sblk = sblk.reshape(B, C * GRP)
    j1 = _extract96(sblk, C * 8, pre_reduce=16, qb=32)
    sub_base = (jnp.take_along_axis(gid2, j1 // 8, axis=1) * GRP
                + (j1 % 8) * 16)
    ecand = _sc_gather_rows(scores.reshape(B * N_PAD),
                            (sub_base[:, :, None] + qrow[:, :, None] * N_PAD
                             + jnp.arange(16, dtype=jnp.int32)).reshape(-1),
                            0, 4096)
    ecand = ecand.reshape(B, C * 16)
    eids = (sub_base[:, :, None]
            + jnp.arange(16, dtype=jnp.int32)).reshape(B, C * 16)
    jp = _extract96(ecand, C * 8, pre_reduce=2)
    epair = _sc_gather_rows(ecand.reshape(B * C * 16),
                            (jp[:, :, None] * 2 + qrow[:, :, None] * (C * 16)
                             + jnp.arange(2, dtype=jnp.int32)).reshape(-1),
                            0, 6144)
    epair = epair.reshape(B, 2 * C)
    pids = jnp.take_along_axis(
        eids, (jp[:, :, None] * 2
               + jnp.arange(2, dtype=jnp.int32)).reshape(B, 2 * C), axis=1)
    j0 = _extract96(epair, 2 * C)
    context_idx = jnp.take_along_axis(pids, j0, axis=1)

    y_pad = jnp.pad(candidate_y, (0, N_PAD - N))
    idx_flat = context_idx.reshape(-1)
    ctx_k, ctx_yflat = _sc_gather(ck, y_pad, idx_flat)
    ctx_y = ctx_yflat[:, None]

    W_head_pad = jnp.pad(W_head, ((0, 0), (0, D_MAIN - 2)))
    b_head_pad = jnp.pad(b_head, (0, D_MAIN - 2))
    out = pl.pallas_call(
        _attn_kernel,
        grid=(B // ATT_QB,),
        in_specs=[
            pl.BlockSpec((ATT_QB, D_MAIN), lambda i: (i, 0)),
            pl.BlockSpec((ATT_QB, D_MAIN), lambda i: (i, 0)),
            pl.BlockSpec((ATT_QB * C, D_MAIN), lambda i: (i, 0)),
            pl.BlockSpec((ATT_QB * C, 1), lambda i: (i, 0)),
            _full((1, D_MAIN)),
            _full((D_MAIN,)),
            _full((D_MAIN, D_BLOCK)),
            _full((D_BLOCK,)),
            _full((D_BLOCK, D_MAIN)),
            _full((D_MAIN,)),
            _full((D_MAIN,)),
            _full((D_MAIN, D_BLOCK)),
            _full((D_BLOCK,)),
            _full((D_BLOCK, D_MAIN)),
            _full((D_MAIN,)),
            _full((D_MAIN,)),
            _full((D_MAIN,)),
            _full((D_MAIN, D_MAIN)),
            _full((D_MAIN,)),
        ],
        out_specs=pl.BlockSpec((ATT_QB, D_MAIN), lambda i: (i, 0)),
        out_shape=jax.ShapeDtypeStruct((B, D_MAIN), f32),
    )(xq, kq, ctx_k, ctx_y, W_label, b_label, W_T1, b_T1, W_T2,
      ln1_g, ln1_b, W_b1, b_b1, W_b2, b_b2, lnh_g, lnh_b,
      W_head_pad, b_head_pad)

    return out[:, :2]


# speedup vs baseline: 6.4541x; 1.1756x over previous
import functools
import jax
import jax.numpy as jnp
from jax import lax
from jax.experimental import pallas as pl
from jax.experimental.pallas import tpu as pltpu
from jax.experimental.pallas import tpu_sc as plsc

B = 1024
N = 100000
D_IN = 64
D_MAIN = 128
D_BLOCK = 256
C = 96
EPS = 1e-5

N_PAD = 100352
ENC_BLK = 2048
SCORE_QB = 128
ATT_QB = 64
NEG = -1e30
GRP = 128
NGRP = N_PAD // GRP


def _encode_cand_kernel(x_ref, wl_ref, bl_ref, wk_ref, bk_ref, ck_ref):
    h = jnp.dot(x_ref[...], wl_ref[...], preferred_element_type=jnp.float32)
    h = h + bl_ref[...]
    ck = jnp.dot(h, wk_ref[...], preferred_element_type=jnp.float32)
    ck_ref[...] = ck + bk_ref[...]


def _encode_query_kernel(x_ref, wl_ref, bl_ref, wk_ref, bk_ref, xq_ref, kq_ref):
    h = jnp.dot(x_ref[...], wl_ref[...], preferred_element_type=jnp.float32)
    h = h + bl_ref[...]
    xq_ref[...] = h
    kq_ref[...] = jnp.dot(h, wk_ref[...], preferred_element_type=jnp.float32) + bk_ref[...]


def _scores_kernel(kq_ref, ck_ref, s_ref, l2_ref):
    j = pl.program_id(1)
    ck = ck_ref[...]
    nsq = jnp.sum(ck * ck, axis=1)[None, :]
    s = 2.0 * jax.lax.dot_general(
        kq_ref[...], ck, (((1,), (1,)), ((), ())),
        preferred_element_type=jnp.float32) - nsq
    col = j * ENC_BLK + lax.broadcasted_iota(jnp.int32, s.shape, 1)
    s = jnp.where(col < N, s, NEG)
    v = lax.bitcast_convert_type(s, jnp.int32)
    key = v ^ ((v >> 31) & jnp.int32(0x7FFFFFFF))
    s_ref[...] = key
    gm = jnp.max(key.reshape(SCORE_QB, ENC_BLK // GRP, GRP), axis=2)
    gcol = lax.broadcasted_iota(jnp.int32, (SCORE_QB, NGRP), 1)
    nbg = ENC_BLK // GRP
    mine = (gcol >= j * nbg) & (gcol < (j + 1) * nbg)
    tiled = jnp.broadcast_to(gm[:, None, :],
                             (SCORE_QB, NGRP // nbg, nbg)).reshape(SCORE_QB, NGRP)
    placed = jnp.where(mine, tiled, jnp.int32(-2147483648))

    @pl.when(j == 0)
    def _():
        l2_ref[...] = placed

    @pl.when(j > 0)
    def _():
        l2_ref[...] = jnp.maximum(l2_ref[...], placed)


def _attn_kernel(xq_ref, kq_ref, ctxk_ref, ctxy_ref,
                 wlab_ref, blab_ref, wt1_ref, bt1_ref, wt2_ref,
                 ln1g_ref, ln1b_ref, wb1_ref, bb1_ref, wb2_ref, bb2_ref,
                 lnhg_ref, lnhb_ref, whead_ref, bhead_ref, out_ref):
    nq = xq_ref.shape[0]
    kq = kq_ref[...]
    ctxk = ctxk_ref[...]
    kq_rows = jnp.repeat(kq, C, axis=0)
    diff = kq_rows - ctxk

    sim = -jnp.sum(diff * diff, axis=1, keepdims=True)
    sim3 = sim.reshape(nq, C, 1)
    m = jnp.max(sim3, axis=1, keepdims=True)
    e = jnp.exp(sim3 - m)
    denom = jnp.sum(e, axis=1, keepdims=True)
    probs = e / denom

    y_emb = ctxy_ref[...] * wlab_ref[...] + blab_ref[...]

    t = jnp.dot(diff, wt1_ref[...], preferred_element_type=jnp.float32) + bt1_ref[...]
    t = jnp.maximum(t, 0.0)
    t = jnp.dot(t, wt2_ref[...], preferred_element_type=jnp.float32)

    values = (y_emb + t).reshape(nq, C, D_MAIN)
    ctx_x = jnp.sum(values * probs, axis=1)

    x = xq_ref[...] + ctx_x

    mu = jnp.mean(x, axis=1, keepdims=True)
    var = jnp.mean((x - mu) ** 2, axis=1, keepdims=True)
    h = (x - mu) * lax.rsqrt(var + EPS) * ln1g_ref[...] + ln1b_ref[...]
    h = jnp.dot(h, wb1_ref[...], preferred_element_type=jnp.float32) + bb1_ref[...]
    h = jnp.maximum(h, 0.0)
    x = x + jnp.dot(h, wb2_ref[...], preferred_element_type=jnp.float32) + bb2_ref[...]

    mu = jnp.mean(x, axis=1, keepdims=True)
    var = jnp.mean((x - mu) ** 2, axis=1, keepdims=True)
    h = (x - mu) * lax.rsqrt(var + EPS) * lnhg_ref[...] + lnhb_ref[...]
    h = jnp.maximum(h, 0.0)
    out_ref[...] = jnp.dot(h, whead_ref[...], preferred_element_type=jnp.float32) + bhead_ref[...]


def _full(shape):
    return pl.BlockSpec(shape, lambda *_: tuple(0 for _ in shape))


def _extract96_kernel(keys_ref, lanes_ref, *, w, pre_reduce):
    v = keys_ref[...]
    nq = v.shape[0]
    if pre_reduce > 1:
        v = jnp.max(v.reshape(nq, w, pre_reduce), axis=2)
    cols = lax.broadcasted_iota(jnp.int32, (nq, w), 1)
    lanes0 = lax.broadcasted_iota(jnp.int32, (nq, 128), 1)
    big = jnp.int32(2147483647)
    neg = jnp.int32(-2147483648)

    def it(t, carry):
        vv, acc = carry
        m = jnp.max(vv, axis=1, keepdims=True)
        cand = jnp.where(vv == m, cols, big)
        j = jnp.min(cand, axis=1, keepdims=True)
        acc = jnp.where(lanes0 == t, j, acc)
        vv = jnp.where(cols == j, neg, vv)
        return (vv, acc)

    _, acc = lax.fori_loop(0, C, it, (v, jnp.zeros((nq, 128), jnp.int32)))
    lanes_ref[...] = acc


def _colmax_kernel(x_ref, o_ref, *, w, r):
    o_ref[...] = jnp.max(x_ref[...].reshape(x_ref.shape[0], w, r), axis=2)


def _colmax(x, w, r, qb=64):
    return pl.pallas_call(
        functools.partial(_colmax_kernel, w=w, r=r),
        grid=(B // qb,),
        in_specs=[pl.BlockSpec((qb, w * r), lambda i: (i, 0))],
        out_specs=pl.BlockSpec((qb, w), lambda i: (i, 0)),
        out_shape=jax.ShapeDtypeStruct((B, w), jnp.int32),
    )(x)


def _extract96(keys, w, pre_reduce=1, qb=128):
    lanes = pl.pallas_call(
        functools.partial(_extract96_kernel, w=w, pre_reduce=pre_reduce),
        grid=(B // qb,),
        in_specs=[pl.BlockSpec((qb, w * pre_reduce), lambda i: (i, 0))],
        out_specs=pl.BlockSpec((qb, 128), lambda i: (i, 0)),
        out_shape=jax.ShapeDtypeStruct((B, 128), jnp.int32),
    )(keys)
    return lanes[:, :C]


def _sc_gather_rows(table, idx, d, chunk):
    m = idx.shape[0]
    info = plsc.get_sparse_core_info()
    nw = info.num_cores * info.num_subcores
    b_per_w = m // nw
    nchunks = b_per_w // chunk
    mesh = plsc.VectorSubcoreMesh(core_axis_name="c", subcore_axis_name="s")
    out_sh = (m, d) if d else (m,)
    row_sh = (chunk, d) if d else (chunk,)

    @functools.partial(
        pl.kernel,
        mesh=mesh,
        out_type=jax.ShapeDtypeStruct(out_sh, table.dtype),
        scratch_types=[
            pltpu.VMEM((b_per_w,), jnp.int32),
            pltpu.VMEM(row_sh, table.dtype),
            pltpu.SemaphoreType.DMA,
        ],
    )
    def k(t_ref, idx_ref, out_ref, idx_v, rows_v, sem):
        wid = lax.axis_index("s") * info.num_cores + lax.axis_index("c")
        base = wid * b_per_w
        pltpu.sync_copy(idx_ref.at[pl.ds(base, b_per_w)], idx_v)

        def body(c, _):
            off = c * chunk
            cidx = idx_v.at[pl.ds(off, chunk)]
            pltpu.async_copy(t_ref.at[cidx], rows_v, sem).wait()
            pltpu.sync_copy(rows_v, out_ref.at[pl.ds(base + off, chunk)])
            return ()

        lax.fori_loop(0, nchunks, body, ())

    return k(table, idx)


GATHER_CHUNK = 384


def _sc_gather(ck_hbm, y_hbm, idx_hbm):
    info = plsc.get_sparse_core_info()
    nw = info.num_cores * info.num_subcores
    b_per_w = (B * C) // nw
    nchunks = b_per_w // GATHER_CHUNK

    mesh = plsc.VectorSubcoreMesh(core_axis_name="c", subcore_axis_name="s")

    @functools.partial(
        pl.kernel,
        mesh=mesh,
        out_type=[jax.ShapeDtypeStruct((B * C, D_MAIN), jnp.float32),
                  jax.ShapeDtypeStruct((B * C,), jnp.float32)],
        scratch_types=[
            pltpu.VMEM((b_per_w,), jnp.int32),
            pltpu.VMEM((GATHER_CHUNK, D_MAIN), jnp.float32),
            pltpu.VMEM((GATHER_CHUNK,), jnp.float32),
            pltpu.SemaphoreType.DMA,
            pltpu.SemaphoreType.DMA,
        ],
    )
    def k(ck_ref, y_ref, idx_ref, outk_ref, outy_ref,
          idx_v, rows_v, yrows_v, sem1, sem2):
        wid = lax.axis_index("s") * info.num_cores + lax.axis_index("c")
        base = wid * b_per_w
        pltpu.sync_copy(idx_ref.at[pl.ds(base, b_per_w)], idx_v)

        def body(c, _):
            off = c * GATHER_CHUNK
            cidx = idx_v.at[pl.ds(off, GATHER_CHUNK)]
            pltpu.async_copy(ck_ref.at[cidx], rows_v, sem1).wait()
            pltpu.sync_copy(rows_v, outk_ref.at[pl.ds(base + off, GATHER_CHUNK)])
            pltpu.async_copy(y_ref.at[cidx], yrows_v, sem2).wait()
            pltpu.sync_copy(yrows_v, outy_ref.at[pl.ds(base + off, GATHER_CHUNK)])
            return ()

        lax.fori_loop(0, nchunks, body, ())

    return k(ck_hbm, y_hbm, idx_hbm)


def kernel(x_num, candidate_x_num, candidate_y, W_lin, b_lin, W_K, b_K,
           W_label, b_label, W_T1, b_T1, W_T2, ln1_g, ln1_b,
           W_b1, b_b1, W_b2, b_b2, lnh_g, lnh_b, W_head, b_head):
    f32 = jnp.float32

    cand_pad = jnp.pad(candidate_x_num, ((0, N_PAD - N), (0, 0)))
    ck = pl.pallas_call(
        _encode_cand_kernel,
        grid=(N_PAD // ENC_BLK,),
        in_specs=[
            pl.BlockSpec((ENC_BLK, D_IN), lambda i: (i, 0)),
            _full((D_IN, D_MAIN)),
            _full((D_MAIN,)),
            _full((D_MAIN, D_MAIN)),
            _full((D_MAIN,)),
        ],
        out_specs=pl.BlockSpec((ENC_BLK, D_MAIN), lambda i: (i, 0)),
        out_shape=jax.ShapeDtypeStruct((N_PAD, D_MAIN), f32),
    )(cand_pad, W_lin, b_lin, W_K, b_K)

    xq, kq = pl.pallas_call(
        _encode_query_kernel,
        grid=(1,),
        in_specs=[
            _full((B, D_IN)),
            _full((D_IN, D_MAIN)),
            _full((D_MAIN,)),
            _full((D_MAIN, D_MAIN)),
            _full((D_MAIN,)),
        ],
        out_specs=[_full((B, D_MAIN)), _full((B, D_MAIN))],
        out_shape=[jax.ShapeDtypeStruct((B, D_MAIN), f32),
                   jax.ShapeDtypeStruct((B, D_MAIN), f32)],
    )(x_num, W_lin, b_lin, W_K, b_K)

    scores, l2max = pl.pallas_call(
        _scores_kernel,
        grid=(B // SCORE_QB, N_PAD // ENC_BLK),
        in_specs=[
            pl.BlockSpec((SCORE_QB, D_MAIN), lambda i, j: (i, 0)),
            pl.BlockSpec((ENC_BLK, D_MAIN), lambda i, j: (j, 0)),
        ],
        out_specs=[
            pl.BlockSpec((SCORE_QB, ENC_BLK), lambda i, j: (i, j)),
            pl.BlockSpec((SCORE_QB, NGRP), lambda i, j: (i, 0)),
        ],
        out_shape=[jax.ShapeDtypeStruct((B, N_PAD), jnp.int32),
                   jax.ShapeDtypeStruct((B, NGRP), jnp.int32)],
    )(kq, ck)

    ar8 = jnp.arange(8, dtype=jnp.int32)
    qrow = jnp.arange(B, dtype=jnp.int32)[:, None]
    gid2 = _extract96(l2max, NGRP)
    sblk = _sc_gather_rows(scores.reshape(B * NGRP, GRP),
                           (gid2 + qrow * NGRP).reshape(-1), GRP, 256)
    sblk = sblk.reshape(B, C * GRP)
    j1 = _extract96(_colmax(sblk, C * 8, 16), C * 8)
    sub_base = (jnp.take_along_axis(gid2, j1 // 8, axis=1) * GRP
                + (j1 % 8) * 16)
    ecand = _sc_gather_rows(scores.reshape(B * N_PAD),
                            (sub_base[:, :, None] + qrow[:, :, None] * N_PAD
                             + jnp.arange(16, dtype=jnp.int32)).reshape(-1),
                            0, 4096)
    ecand = ecand.reshape(B, C * 16)
    eids = (sub_base[:, :, None]
            + jnp.arange(16, dtype=jnp.int32)).reshape(B, C * 16)
    jp = _extract96(ecand, C * 8, pre_reduce=2)
    epair = _sc_gather_rows(ecand.reshape(B * C * 16),
                            (jp[:, :, None] * 2 + qrow[:, :, None] * (C * 16)
                             + jnp.arange(2, dtype=jnp.int32)).reshape(-1),
                            0, 6144)
    epair = epair.reshape(B, 2 * C)
    pids = jnp.take_along_axis(
        eids, (jp[:, :, None] * 2
               + jnp.arange(2, dtype=jnp.int32)).reshape(B, 2 * C), axis=1)
    j0 = _extract96(epair, 2 * C)
    context_idx = jnp.take_along_axis(pids, j0, axis=1)

    y_pad = jnp.pad(candidate_y, (0, N_PAD - N))
    idx_flat = context_idx.reshape(-1)
    ctx_k, ctx_yflat = _sc_gather(ck, y_pad, idx_flat)
    ctx_y = ctx_yflat[:, None]

    W_head_pad = jnp.pad(W_head, ((0, 0), (0, D_MAIN - 2)))
    b_head_pad = jnp.pad(b_head, (0, D_MAIN - 2))
    out = pl.pallas_call(
        _attn_kernel,
        grid=(B // ATT_QB,),
        in_specs=[
            pl.BlockSpec((ATT_QB, D_MAIN), lambda i: (i, 0)),
            pl.BlockSpec((ATT_QB, D_MAIN), lambda i: (i, 0)),
            pl.BlockSpec((ATT_QB * C, D_MAIN), lambda i: (i, 0)),
            pl.BlockSpec((ATT_QB * C, 1), lambda i: (i, 0)),
            _full((1, D_MAIN)),
            _full((D_MAIN,)),
            _full((D_MAIN, D_BLOCK)),
            _full((D_BLOCK,)),
            _full((D_BLOCK, D_MAIN)),
            _full((D_MAIN,)),
            _full((D_MAIN,)),
            _full((D_MAIN, D_BLOCK)),
            _full((D_BLOCK,)),
            _full((D_BLOCK, D_MAIN)),
            _full((D_MAIN,)),
            _full((D_MAIN,)),
            _full((D_MAIN,)),
            _full((D_MAIN, D_MAIN)),
            _full((D_MAIN,)),
        ],
        out_specs=pl.BlockSpec((ATT_QB, D_MAIN), lambda i: (i, 0)),
        out_shape=jax.ShapeDtypeStruct((B, D_MAIN), f32),
    )(xq, kq, ctx_k, ctx_y, W_label, b_label, W_T1, b_T1, W_T2,
      ln1_g, ln1_b, W_b1, b_b1, W_b2, b_b2, lnh_g, lnh_b,
      W_head_pad, b_head_pad)

    return out[:, :2]
